# baseline (speedup 1.0000x reference)
import functools
import math

import jax
import jax.numpy as jnp
from jax import lax
from jax.experimental import pallas as pl
from jax.experimental.pallas import tpu as pltpu
from jax.experimental.pallas import tpu_sc as plsc

D_MODEL = 64
SCALE = math.sqrt(D_MODEL)
NUM_CORES = 2
NUM_SUBCORES = 16
NUM_WORKERS = NUM_CORES * NUM_SUBCORES

CHUNK = 128
NBUF = 4
AHEAD = NBUF - 1


def _gather(tokens_flat, table, B, S, SEQ):
    mesh = plsc.VectorSubcoreMesh(core_axis_name="c", subcore_axis_name="s")
    b_per_w = B // NUM_WORKERS
    n_chunks = b_per_w // CHUNK

    @functools.partial(
        pl.kernel,
        out_type=jax.ShapeDtypeStruct((SEQ, D_MODEL, S), jnp.float32),
        mesh=mesh,
        scratch_types=[
            pltpu.VMEM((NBUF, CHUNK), jnp.int32),
            pltpu.VMEM((NBUF, CHUNK, D_MODEL), jnp.float32),
            pltpu.VMEM((NBUF, D_MODEL, CHUNK), jnp.float32),
        ]
        + [pltpu.SemaphoreType.DMA] * (2 * NBUF),
        compiler_params=pltpu.CompilerParams(
            use_tc_tiling_on_sc=False, needs_layout_passes=False
        ),
    )
    def body(tok_hbm, tab_hbm, out_hbm, idx_v, rows_v, obuf_v, *sems):
        gsem = sems[:NBUF]
        ssem = sems[NBUF:]
        wid = lax.axis_index("s") * NUM_CORES + lax.axis_index("c")
        base = wid * b_per_w

        def issue_gather(g, slot):
            off = base + g * CHUNK
            pltpu.sync_copy(tok_hbm.at[pl.ds(off, CHUNK)], idx_v.at[slot])
            pltpu.async_copy(
                tab_hbm.at[idx_v.at[slot]], rows_v.at[slot], gsem[slot]
            )

        for g in range(AHEAD):
            issue_gather(g, g % NBUF)

        def outer(t, carry):
            for j in range(NBUF):
                g = t * NBUF + j
                pltpu.make_async_copy(
                    tab_hbm.at[idx_v.at[j]], rows_v.at[j], gsem[j]
                ).wait()

                @pl.when(g >= NBUF)
                def _():
                    pltpu.make_async_copy(
                        obuf_v.at[j],
                        out_hbm.at[0, :, pl.ds(0, CHUNK)],
                        ssem[j],
                    ).wait()

                @plsc.parallel_loop(0, D_MODEL, 1, unroll=4)
                def _(d):
                    cols = jnp.full((16,), d, jnp.int32)
                    for q in range(CHUNK // 16):
                        rows16 = jax.lax.iota(jnp.int32, 16) + (q * 16)
                        vals = plsc.load_gather(rows_v.at[j], [rows16, cols])
                        obuf_v[j, d, pl.ds(q * 16, 16)] = vals * SCALE

                off = base + g * CHUNK
                s = off // S
                b0 = off - s * S
                pltpu.async_copy(
                    obuf_v.at[j],
                    out_hbm.at[s, :, pl.ds(b0, CHUNK)],
                    ssem[j],
                )

                nxt = g + AHEAD

                @pl.when(nxt < n_chunks)
                def _():
                    issue_gather(nxt, (j + AHEAD) % NBUF)

            return carry

        lax.fori_loop(0, n_chunks // NBUF, outer, 0)

        for j in range(NBUF):
            pltpu.make_async_copy(
                obuf_v.at[j], out_hbm.at[0, :, pl.ds(0, CHUNK)], ssem[j]
            ).wait()

    return body(tokens_flat, table)


def kernel(tokens, table):
    S, SEQ = tokens.shape
    B = S * SEQ
    tok_flat = jnp.transpose(tokens).reshape(B).astype(jnp.int32)
    out_t = _gather(tok_flat, table, B, S, SEQ)
    return jnp.transpose(out_t, (2, 0, 1))


# speedup vs baseline: 1.1292x; 1.1292x over previous
import functools
import math

import jax
import jax.numpy as jnp
from jax import lax
from jax.experimental import pallas as pl
from jax.experimental.pallas import tpu as pltpu
from jax.experimental.pallas import tpu_sc as plsc

D_MODEL = 64
SCALE = math.sqrt(D_MODEL)
NUM_CORES = 2
NUM_SUBCORES = 16
NUM_WORKERS = NUM_CORES * NUM_SUBCORES

CHUNK = 128
NBUF = 4
AHEAD = NBUF - 1


def _gather(tokens_flat, table, B, S, SEQ):
    mesh = plsc.VectorSubcoreMesh(core_axis_name="c", subcore_axis_name="s")
    b_per_w = B // NUM_WORKERS
    n_chunks = b_per_w // CHUNK

    @functools.partial(
        pl.kernel,
        out_type=jax.ShapeDtypeStruct(
            (SEQ, D_MODEL // 8, S // CHUNK, 8, CHUNK), jnp.float32
        ),
        mesh=mesh,
        scratch_types=[
            pltpu.VMEM((NBUF, CHUNK), jnp.int32),
            pltpu.VMEM((NBUF, D_MODEL // 8, 8, CHUNK), jnp.float32),
            pltpu.VMEM((NBUF, CHUNK, D_MODEL), jnp.float32),
        ]
        + [pltpu.SemaphoreType.DMA] * (2 * NBUF),
        compiler_params=pltpu.CompilerParams(
            use_tc_tiling_on_sc=False, needs_layout_passes=False
        ),
    )
    def body(tok_hbm, tab_hbm, out_hbm, idx_v, obuf_v, rows_v, *sems):
        gsem = sems[:NBUF]
        ssem = sems[NBUF:]
        wid = lax.axis_index("s") * NUM_CORES + lax.axis_index("c")
        base = wid * b_per_w

        def issue_gather(g, slot):
            off = base + g * CHUNK
            pltpu.sync_copy(tok_hbm.at[pl.ds(off, CHUNK)], idx_v.at[slot])
            pltpu.async_copy(
                tab_hbm.at[idx_v.at[slot]], rows_v.at[slot], gsem[slot]
            )

        for g in range(AHEAD):
            issue_gather(g, g % NBUF)

        def outer(t, carry):
            for j in range(NBUF):
                g = t * NBUF + j
                pltpu.make_async_copy(
                    tab_hbm.at[idx_v.at[j]], rows_v.at[j], gsem[j]
                ).wait()

                @pl.when(g >= NBUF)
                def _():
                    pltpu.make_async_copy(
                        obuf_v.at[j],
                        out_hbm.at[0, :, 0],
                        ssem[j],
                    ).wait()

                @plsc.parallel_loop(0, D_MODEL, 1, unroll=8)
                def _(d):
                    cols = jnp.full((16,), d, jnp.int32)
                    for q in range(CHUNK // 16):
                        rows16 = jax.lax.iota(jnp.int32, 16) + (q * 16)
                        vals = plsc.load_gather(rows_v.at[j], [rows16, cols])
                        obuf_v[j, d // 8, d % 8, pl.ds(q * 16, 16)] = (
                            vals * SCALE
                        )

                off = base + g * CHUNK
                s = off // S
                bt = (off - s * S) // CHUNK
                pltpu.async_copy(
                    obuf_v.at[j],
                    out_hbm.at[s, :, bt],
                    ssem[j],
                )

                nxt = g + AHEAD

                @pl.when(nxt < n_chunks)
                def _():
                    issue_gather(nxt, (j + AHEAD) % NBUF)

            return carry

        lax.fori_loop(0, n_chunks // NBUF, outer, 0)

        for j in range(NBUF):
            pltpu.make_async_copy(
                obuf_v.at[j], out_hbm.at[0, :, 0], ssem[j]
            ).wait()

    return body(tokens_flat, table)


def kernel(tokens, table):
    S, SEQ = tokens.shape
    B = S * SEQ
    tok_flat = jnp.transpose(tokens).reshape(B).astype(jnp.int32)
    out5 = _gather(tok_flat, table, B, S, SEQ)
    return jnp.transpose(out5, (2, 4, 0, 1, 3)).reshape(S, SEQ, D_MODEL)
